# baseline (speedup 1.0000x reference)
import jax
import jax.numpy as jnp
from jax import lax
from jax.experimental import pallas as pl
from jax.experimental.pallas import tpu as pltpu

_IMG = 128.0
_CONF = 0.25
_IOU = 0.45
_NC = 9
_NCAND = 300
_CPAD = 384
_E = 20
_N = 20000
_NPAD = 20480
_ROWS = 160
_HI = jax.lax.Precision.HIGHEST


def _fiota(shape, dim):
    return lax.broadcasted_iota(jnp.int32, shape, dim).astype(jnp.float32)


def _body(pred_ref, out_ref, iou_ref, sc_ref):
    p = pred_ref[0]
    cx, cy, w, h, obj = p[0], p[1], p[2], p[3], p[4]

    best = obj * p[5]
    bcls = jnp.zeros_like(best)
    for k in range(1, _NC):
        c = obj * p[5 + k]
        upd = c > best
        bcls = jnp.where(upd, jnp.float32(k), bcls)
        best = jnp.where(upd, c, best)

    x1p = cx - w * 0.5
    y1p = cy - h * 0.5
    x2p = cx + w * 0.5
    y2p = cy + h * 0.5

    scores = jnp.where(best > _CONF, best, -1.0)

    lan = _fiota((1, _CPAD), 1)
    idx2d = (_fiota((_ROWS, 128), 0) * 128.0
             + _fiota((_ROWS, 128), 1))

    def _repmax(v):
        for s in (64, 32, 16, 8, 4, 2, 1):
            v = jnp.maximum(v, pltpu.roll(v, s, 1))
        return v

    def _repmin(v):
        for s in (64, 32, 16, 8, 4, 2, 1):
            v = jnp.minimum(v, pltpu.roll(v, s, 1))
        return v

    def _wide(v):
        return jnp.concatenate([v] * (_CPAD // 128), axis=1)

    def sel_body(t, carry):
        sc, tmf, selr, selc, sels = carry
        m = _repmax(jnp.max(sc, axis=0, keepdims=True))
        pick = _repmin(jnp.min(jnp.where(sc == m, idx2d, 3.0e7),
                               axis=0, keepdims=True))
        sc = jnp.where(idx2d == pick, -2.0, sc)
        r = jnp.floor(pick * (1.0 / 128.0))
        cc = pick - r * 128.0
        selr = selr + tmf * _wide(r)
        selc = selc + tmf * _wide(cc)
        sels = sels + tmf * _wide(m)
        return sc, pltpu.roll(tmf, 1, 1), selr, selc, sels

    z = jnp.zeros((1, _CPAD), jnp.float32)
    tmf0 = (lan == 0.0).astype(jnp.float32)
    _, _, selr, selc, sels = lax.fori_loop(
        0, _NCAND, sel_body, (scores, tmf0, z, z, z))

    rowio = _fiota((_ROWS, _CPAD), 0)
    a_t = (rowio == selr).astype(jnp.float32)
    feats = jnp.concatenate([x1p, y1p, x2p, y2p, bcls], axis=1)
    gath = lax.dot_general(feats, a_t, (((0,), (0,)), ((), ())),
                           precision=_HI)
    cio = _fiota((128, _CPAD), 0)
    colm = (cio == selc).astype(jnp.float32)

    def pickf(i):
        return jnp.sum(gath[128 * i:128 * (i + 1), :] * colm,
                       axis=0, keepdims=True)

    x1v, y1v, x2v, y2v, clsv = (pickf(i) for i in range(5))
    confv = sels

    off = clsv * (_IMG * 64.0)
    ox1, oy1, ox2, oy2 = x1v + off, y1v + off, x2v + off, y2v + off
    area = (ox2 - ox1) * (oy2 - oy1)

    subio = _fiota((_CPAD, _CPAD), 0)
    lanio = _fiota((_CPAD, _CPAD), 1)
    ident = (subio == lanio).astype(jnp.float32)

    def tcol(v):
        return lax.dot_general(ident, v, (((1,), (1,)), ((), ())),
                               precision=_HI)

    ox1c, oy1c, ox2c, oy2c, areac = (tcol(v) for v in
                                     (ox1, oy1, ox2, oy2, area))

    xx1 = jnp.maximum(ox1c, ox1)
    yy1 = jnp.maximum(oy1c, oy1)
    xx2 = jnp.minimum(ox2c, ox2)
    yy2 = jnp.minimum(oy2c, oy2)
    iw = jnp.clip(xx2 - xx1, 0.0, None)
    ih = jnp.clip(yy2 - yy1, 0.0, None)
    inter = iw * ih
    iou_ref[...] = inter / (areac + area - inter + 1e-9)

    keep0 = (confv > _CONF).astype(jnp.float32)

    nk = 16
    lanK = _fiota((nk, _CPAD), 1)
    subK = _fiota((nk, _CPAD), 0)

    def nms_chunk(b, keepv):
        c0 = b * nk
        c0f = c0.astype(jnp.float32)
        blk = iou_ref[pl.ds(c0, nk), :]
        rowf = subK + c0f
        supblk = (blk > _IOU) & (lanK > rowf)
        eqblk = (lanK == rowf).astype(jnp.float32)
        for k in range(nk):
            keep_i = jnp.max(eqblk[k:k + 1, :] * keepv)
            sup = supblk[k:k + 1, :] & (keep_i > 0.0)
            keepv = jnp.where(sup, 0.0, keepv)
        return keepv

    keepv = lax.fori_loop(0, (_NCAND + nk - 1) // nk, nms_chunk, keep0)

    tri = (subio <= lanio).astype(jnp.float32)
    rank = lax.dot_general(keepv, tri, (((1,), (0,)), ((), ())),
                           precision=_HI) - 1.0
    eio = _fiota((_E, _CPAD), 0)
    oneh = ((rank == eio) & (keepv > 0.0)).astype(jnp.float32)

    def pack(v):
        return lax.dot_general(oneh, v, (((1,), (1,)), ((), ())),
                               precision=_HI)

    x1o, y1o, x2o, y2o, clso, probo = (pack(v) for v in
                                       (x1v, y1v, x2v, y2v, clsv, confv))

    cgrp = jnp.floor(clso / 3.0)
    sgrp = clso - 3.0 * cgrp
    cio3 = _fiota((_E, 3), 1)
    color = jnp.where(cgrp == cio3, probo, 0.0)
    shp = jnp.where(sgrp == cio3, probo, 0.0)
    xy = jnp.concatenate([x1o, y1o, x2o, y2o], axis=1) * (1.0 / _IMG)
    out_ref[0] = jnp.concatenate([xy, color, shp, probo], axis=1)


@jax.jit
def kernel(pred):
    b = pred.shape[0]
    pp = jnp.pad(pred, ((0, 0), (0, _NPAD - _N), (0, 0)))
    pp = pp.transpose(0, 2, 1).reshape(b, 14, _ROWS, 128)
    return pl.pallas_call(
        _body,
        grid=(b,),
        in_specs=[pl.BlockSpec((1, 14, _ROWS, 128), lambda i: (i, 0, 0, 0))],
        out_specs=pl.BlockSpec((1, _E, 11), lambda i: (i, 0, 0)),
        out_shape=jax.ShapeDtypeStruct((b, _E, 11), jnp.float32),
        scratch_shapes=[pltpu.VMEM((_CPAD, _CPAD), jnp.float32),
                        pltpu.VMEM((_ROWS, 128), jnp.float32)],
    )(pp)


# speedup vs baseline: 5.1507x; 5.1507x over previous
import jax
import jax.numpy as jnp
from jax import lax
from jax.experimental import pallas as pl
from jax.experimental.pallas import tpu as pltpu

_IMG = 128.0
_CONF = 0.25
_IOU = 0.45
_NC = 9
_NCAND = 300
_CPAD = 384
_E = 20
_N = 20000
_NPAD = 20480
_ROWS = 160
_B = 4
_HI = jax.lax.Precision.HIGHEST


def _fiota(shape, dim):
    return lax.broadcasted_iota(jnp.int32, shape, dim).astype(jnp.float32)


def _body(pred_ref, out_ref, iou_ref):
    lan = _fiota((1, _CPAD), 1)
    idx2d = (_fiota((_ROWS, 128), 0) * 128.0
             + _fiota((_ROWS, 128), 1))

    scores, x1s, y1s, x2s, y2s, clss = [], [], [], [], [], []
    for b in range(_B):
        p = pred_ref[b]
        cx, cy, w, h, obj = p[0], p[1], p[2], p[3], p[4]
        best = obj * p[5]
        bcls = jnp.zeros_like(best)
        for k in range(1, _NC):
            c = obj * p[5 + k]
            upd = c > best
            bcls = jnp.where(upd, jnp.float32(k), bcls)
            best = jnp.where(upd, c, best)
        x1s.append(cx - w * 0.5)
        y1s.append(cy - h * 0.5)
        x2s.append(cx + w * 0.5)
        y2s.append(cy + h * 0.5)
        clss.append(bcls)
        scores.append(jnp.where(best > _CONF, best, -1.0))

    def sel_body(t, carry):
        scs, selrs, selcs, selss = carry
        tm = lan == t.astype(jnp.float32)
        out_sc, out_r, out_c, out_s = [], [], [], []
        for b in range(_B):
            sc = scs[b]
            m = jnp.max(sc)
            pick = jnp.min(jnp.where(sc == m, idx2d, 3.0e7))
            sc = jnp.where(idx2d == pick, -2.0, sc)
            r = jnp.floor(pick * (1.0 / 128.0))
            cc = pick - r * 128.0
            out_sc.append(sc)
            out_r.append(jnp.where(tm, r, selrs[b]))
            out_c.append(jnp.where(tm, cc, selcs[b]))
            out_s.append(jnp.where(tm, m, selss[b]))
        return tuple(out_sc), tuple(out_r), tuple(out_c), tuple(out_s)

    z = jnp.zeros((1, _CPAD), jnp.float32)
    z4 = (z,) * _B
    neg4 = (jnp.full((1, _CPAD), -2.0),) * _B
    _, selrs, selcs, selss = lax.fori_loop(
        0, _NCAND, sel_body, (tuple(scores), z4, z4, neg4))

    rowio = _fiota((_ROWS, _CPAD), 0)
    cio = _fiota((128, _CPAD), 0)
    subio = _fiota((_CPAD, _CPAD), 0)
    lanio = _fiota((_CPAD, _CPAD), 1)
    ident = (subio == lanio).astype(jnp.float32)
    tri = (subio <= lanio).astype(jnp.float32)
    nk = 16
    lanK = _fiota((nk, _CPAD), 1)
    subK = _fiota((nk, _CPAD), 0)
    eio = _fiota((_E, _CPAD), 0)
    cio3 = _fiota((_E, 3), 1)

    def tcol(v):
        return lax.dot_general(ident, v, (((1,), (1,)), ((), ())),
                               precision=_HI)

    feats_v = []
    for b in range(_B):
        a_t = (rowio == selrs[b]).astype(jnp.float32)
        feats = jnp.concatenate(
            [x1s[b], y1s[b], x2s[b], y2s[b], clss[b]], axis=1)
        gath = lax.dot_general(feats, a_t, (((0,), (0,)), ((), ())),
                               precision=_HI)
        colm = (cio == selcs[b]).astype(jnp.float32)

        def pickf(i, g=gath, cm=colm):
            return jnp.sum(g[128 * i:128 * (i + 1), :] * cm,
                           axis=0, keepdims=True)

        x1v, y1v, x2v, y2v, clsv = (pickf(i) for i in range(5))
        feats_v.append((x1v, y1v, x2v, y2v, clsv))

        off = clsv * (_IMG * 64.0)
        ox1, oy1, ox2, oy2 = x1v + off, y1v + off, x2v + off, y2v + off
        area = (ox2 - ox1) * (oy2 - oy1)
        xx1 = jnp.maximum(tcol(ox1), ox1)
        yy1 = jnp.maximum(tcol(oy1), oy1)
        xx2 = jnp.minimum(tcol(ox2), ox2)
        yy2 = jnp.minimum(tcol(oy2), oy2)
        iw = jnp.clip(xx2 - xx1, 0.0, None)
        ih = jnp.clip(yy2 - yy1, 0.0, None)
        inter = iw * ih
        iou_ref[b] = inter / (tcol(area) + area - inter + 1e-9)

    keeps = tuple((selss[b] > _CONF).astype(jnp.float32) for b in range(_B))

    def nms_chunk(cb, keeps):
        c0 = cb * nk
        c0f = c0.astype(jnp.float32)
        rowf = subK + c0f
        ltri = lanK > rowf
        eqblk = (lanK == rowf).astype(jnp.float32)
        sups = [(iou_ref[b, pl.ds(c0, nk), :] > _IOU) & ltri
                for b in range(_B)]
        for k in range(nk):
            eq = eqblk[k:k + 1, :]
            new = []
            for b in range(_B):
                keep_i = jnp.max(eq * keeps[b])
                sup = sups[b][k:k + 1, :] & (keep_i > 0.0)
                new.append(jnp.where(sup, 0.0, keeps[b]))
            keeps = tuple(new)
        return keeps

    keeps = lax.fori_loop(0, (_NCAND + nk - 1) // nk, nms_chunk, keeps)

    for b in range(_B):
        keepv = keeps[b]
        rank = lax.dot_general(keepv, tri, (((1,), (0,)), ((), ())),
                               precision=_HI) - 1.0
        oneh = ((rank == eio) & (keepv > 0.0)).astype(jnp.float32)

        def pack(v, oh=oneh):
            return lax.dot_general(oh, v, (((1,), (1,)), ((), ())),
                                   precision=_HI)

        x1v, y1v, x2v, y2v, clsv = feats_v[b]
        x1o, y1o, x2o, y2o, clso, probo = (
            pack(v) for v in (x1v, y1v, x2v, y2v, clsv, selss[b]))

        cgrp = jnp.floor(clso / 3.0)
        sgrp = clso - 3.0 * cgrp
        color = jnp.where(cgrp == cio3, probo, 0.0)
        shp = jnp.where(sgrp == cio3, probo, 0.0)
        xy = jnp.concatenate([x1o, y1o, x2o, y2o], axis=1) * (1.0 / _IMG)
        out_ref[b] = jnp.concatenate([xy, color, shp, probo], axis=1)


@jax.jit
def kernel(pred):
    b = pred.shape[0]
    pp = jnp.pad(pred, ((0, 0), (0, _NPAD - _N), (0, 0)))
    pp = pp.transpose(0, 2, 1).reshape(b, 14, _ROWS, 128)
    return pl.pallas_call(
        _body,
        out_shape=jax.ShapeDtypeStruct((b, _E, 11), jnp.float32),
        scratch_shapes=[pltpu.VMEM((_B, _CPAD, _CPAD), jnp.float32)],
    )(pp)
